# baseline (speedup 1.0000x reference)
import functools

import jax
import jax.numpy as jnp
from jax import lax
from jax.experimental import pallas as pl
from jax.experimental.pallas import tpu as pltpu
from jax.experimental.pallas import tpu_sc as plsc

POOL = 7
C = 768
H = 56
W = 56
B = 2
N = 16
LANES = 16
YW = 48
GK = 16
CS = 256
CTC = C - CS
WIN = 16
WINY = 24

_GROUPS = []
_base = 0
while _base < CS:
    _nv = min(GK, (CS - _base) // LANES)
    _GROUPS.append((_base, _nv))
    _base += _nv * LANES


def _sc_body(feat_hbm, rois_hbm, out_hbm, rois_v, row_v, acc_v, sem0, sem1):
    b = lax.axis_index("c")
    n = lax.axis_index("s")
    wid = b * N + n

    pltpu.sync_copy(rois_hbm.at[pl.ds(wid * LANES, LANES)], rois_v)
    vf = rois_v[...]

    def _lane(j):
        return vf[j].astype(jnp.int32)

    min_x, min_y, max_x, max_y = _lane(0), _lane(1), _lane(2), _lane(3)
    dx = (max_x - min_x) // POOL
    dy = (max_y - min_y) // POOL

    y0 = jnp.minimum((min_y // 8) * 8, jnp.int32(W - YW))
    dmy = min_y - y0

    neg_inf = jnp.full((LANES,), -jnp.inf, jnp.float32)

    for h in range(POOL):
        for w in range(POOL):

            def _init(i, carry, h=h, w=w):
                acc_v[h, w, pl.ds(i * LANES, LANES)] = neg_inf
                return carry

            lax.fori_loop(0, CS // LANES, _init, jnp.int32(0))

    sems = (sem0, sem1)

    def _start(x, p):
        pltpu.async_copy(
            feat_hbm.at[b, x, pl.ds(y0, YW), pl.ds(0, CS)],
            row_v.at[p],
            sems[p],
        )

    def _wait(p):
        pltpu.make_async_copy(
            feat_hbm.at[0, 0, pl.ds(0, YW), pl.ds(0, CS)],
            row_v.at[p],
            sems[p],
        ).wait()

    def _compute(x, p):
        xr = x - min_x
        w_idx = jnp.int32(0)
        for k in range(1, POOL):
            w_idx = w_idx + (xr >= k * dx).astype(jnp.int32)
        for h in range(POOL):
            o1 = dmy + h * dy
            o2 = dmy + ((h + 1) * dy if h + 1 < POOL else max_y - min_y)
            for gbase, nv in _GROUPS:
                carries = tuple(
                    acc_v[h, w_idx, pl.ds(gbase + j * LANES, LANES)]
                    for j in range(nv)
                )

                def _ybody(y, cs2, gbase=gbase, nv=nv):
                    return tuple(
                        jnp.maximum(
                            cs2[j], row_v[p, y, pl.ds(gbase + j * LANES, LANES)]
                        )
                        for j in range(nv)
                    )

                carries = lax.fori_loop(o1, o2, _ybody, carries)
                for j in range(nv):
                    acc_v[h, w_idx, pl.ds(gbase + j * LANES, LANES)] = carries[j]

    nx = max_x - min_x
    _start(min_x, 0)

    def _pair(k, carry):
        x0 = min_x + 2 * k
        has1 = x0 + 1 < max_x

        @pl.when(has1)
        def _():
            _start(x0 + 1, 1)

        _wait(0)
        _compute(x0, 0)

        @pl.when(has1)
        def _():
            @pl.when(x0 + 2 < max_x)
            def _():
                _start(x0 + 2, 0)

            _wait(1)
            _compute(x0 + 1, 1)

        return carry

    lax.fori_loop(0, (nx + 1) // 2, _pair, jnp.int32(0))

    pltpu.sync_copy(acc_v, out_hbm.at[b, n])


_mesh = plsc.VectorSubcoreMesh(core_axis_name="c", subcore_axis_name="s")

_sc_pool = functools.partial(
    pl.kernel,
    mesh=_mesh,
    out_type=jax.ShapeDtypeStruct((B, N, POOL, POOL, CS), jnp.float32),
    scratch_types=[
        pltpu.VMEM((LANES,), jnp.float32),
        pltpu.VMEM((2, YW, CS), jnp.float32),
        pltpu.VMEM((POOL, POOL, CS), jnp.float32),
        pltpu.SemaphoreType.DMA,
        pltpu.SemaphoreType.DMA,
    ],
)(_sc_body)


def _tc_body(rois_ref, feat_ref, out_ref):
    b = pl.program_id(0)
    n = pl.program_id(1)

    def rd(k):
        return rois_ref[b, n, k]

    min_x, min_y, max_x, max_y = rd(0), rd(1), rd(2), rd(3)
    dx = (max_x - min_x) // POOL
    dy = (max_y - min_y) // POOL

    for h in range(POOL):
        wy = 12 if h + 1 < POOL else 18
        y1 = min_y + h * dy
        y2 = min_y + (h + 1) * dy if h + 1 < POOL else max_y
        y1c = jnp.minimum((y1 // 8) * 8, jnp.int32(W - wy))
        y1c = pl.multiple_of(y1c, 8)
        for w in range(POOL):
            wx = 5 if w + 1 < POOL else 10
            x1 = min_x + w * dx
            x2 = min_x + (w + 1) * dx if w + 1 < POOL else max_x
            x1c = jnp.minimum(x1, jnp.int32(H - wx))
            win = feat_ref[0, pl.ds(x1c, wx), pl.ds(y1c, wy), CS:]
            xg = lax.broadcasted_iota(jnp.int32, (wx, wy, 1), 0) + x1c
            yg = lax.broadcasted_iota(jnp.int32, (wx, wy, 1), 1) + y1c
            m = (xg >= x1) & (xg < x2) & (yg >= y1) & (yg < y2)
            cell = jnp.max(jnp.where(m, win, -jnp.inf), axis=(0, 1))
            out_ref[0, 0, h, w, :] = cell


_tc_pool = pl.pallas_call(
    _tc_body,
    grid_spec=pltpu.PrefetchScalarGridSpec(
        num_scalar_prefetch=1,
        grid=(B, N),
        in_specs=[
            pl.BlockSpec((1, H, W, C), lambda b, n, rois: (b, 0, 0, 0)),
        ],
        out_specs=pl.BlockSpec(
            (1, 1, POOL, POOL, CTC), lambda b, n, rois: (b, n, 0, 0, 0)
        ),
    ),
    out_shape=jax.ShapeDtypeStruct((B, N, POOL, POOL, CTC), jnp.float32),
)


def kernel(features, rois):
    rois_pad = jnp.zeros((B * N, LANES), jnp.float32)
    rois_pad = rois_pad.at[:, :4].set(rois.reshape(B * N, 4)).reshape(-1)
    out_sc = _sc_pool(features, rois_pad)
    out_tc = _tc_pool(rois.astype(jnp.int32), features)
    return jnp.concatenate([out_sc, out_tc], axis=-1)


# speedup vs baseline: 1.0612x; 1.0612x over previous
import functools

import jax
import jax.numpy as jnp
from jax import lax
from jax.experimental import pallas as pl
from jax.experimental.pallas import tpu as pltpu
from jax.experimental.pallas import tpu_sc as plsc

POOL = 7
C = 768
H = 56
W = 56
B = 2
N = 16
LANES = 16
YW = 48
GK = 16
CS = 256
CTC = C - CS
WIN = 16
WINY = 24

_GROUPS = []
_base = 0
while _base < CS:
    _nv = min(GK, (CS - _base) // LANES)
    _GROUPS.append((_base, _nv))
    _base += _nv * LANES


def _sc_body(feat_hbm, rois_hbm, out_hbm, rois_v, row_v, acc_v, sem0, sem1):
    b = lax.axis_index("c")
    n = lax.axis_index("s")
    wid = b * N + n

    pltpu.sync_copy(rois_hbm.at[pl.ds(wid * LANES, LANES)], rois_v)
    vf = rois_v[...]

    def _lane(j):
        return vf[j].astype(jnp.int32)

    min_x, min_y, max_x, max_y = _lane(0), _lane(1), _lane(2), _lane(3)
    dx = (max_x - min_x) // POOL
    dy = (max_y - min_y) // POOL

    y0 = jnp.minimum((min_y // 8) * 8, jnp.int32(W - YW))
    dmy = min_y - y0

    neg_inf = jnp.full((LANES,), -jnp.inf, jnp.float32)

    for h in range(POOL):
        for w in range(POOL):

            def _init(i, carry, h=h, w=w):
                acc_v[h, w, pl.ds(i * LANES, LANES)] = neg_inf
                return carry

            lax.fori_loop(0, CS // LANES, _init, jnp.int32(0))

    sems = (sem0, sem1)

    def _start(x, p):
        pltpu.async_copy(
            feat_hbm.at[b, x, pl.ds(y0, YW), pl.ds(0, CS)],
            row_v.at[p],
            sems[p],
        )

    def _wait(p):
        pltpu.make_async_copy(
            feat_hbm.at[0, 0, pl.ds(0, YW), pl.ds(0, CS)],
            row_v.at[p],
            sems[p],
        ).wait()

    def _compute(x, p):
        xr = x - min_x
        w_idx = jnp.int32(0)
        for k in range(1, POOL):
            w_idx = w_idx + (xr >= k * dx).astype(jnp.int32)
        for h in range(POOL):
            o1 = dmy + h * dy
            o2 = dmy + ((h + 1) * dy if h + 1 < POOL else max_y - min_y)
            for gbase, nv in _GROUPS:
                carries = tuple(
                    acc_v[h, w_idx, pl.ds(gbase + j * LANES, LANES)]
                    for j in range(nv)
                )

                def _ybody(y, cs2, gbase=gbase, nv=nv):
                    return tuple(
                        jnp.maximum(
                            cs2[j], row_v[p, y, pl.ds(gbase + j * LANES, LANES)]
                        )
                        for j in range(nv)
                    )

                carries = lax.fori_loop(o1, o2, _ybody, carries)
                for j in range(nv):
                    acc_v[h, w_idx, pl.ds(gbase + j * LANES, LANES)] = carries[j]

    nx = max_x - min_x
    _start(min_x, 0)

    def _pair(k, carry):
        x0 = min_x + 2 * k
        has1 = x0 + 1 < max_x

        @pl.when(has1)
        def _():
            _start(x0 + 1, 1)

        _wait(0)
        _compute(x0, 0)

        @pl.when(has1)
        def _():
            @pl.when(x0 + 2 < max_x)
            def _():
                _start(x0 + 2, 0)

            _wait(1)
            _compute(x0 + 1, 1)

        return carry

    lax.fori_loop(0, (nx + 1) // 2, _pair, jnp.int32(0))

    pltpu.sync_copy(acc_v, out_hbm.at[b, n])


_mesh = plsc.VectorSubcoreMesh(core_axis_name="c", subcore_axis_name="s")

_sc_pool = functools.partial(
    pl.kernel,
    mesh=_mesh,
    out_type=jax.ShapeDtypeStruct((B, N, POOL, POOL, CS), jnp.float32),
    scratch_types=[
        pltpu.VMEM((LANES,), jnp.float32),
        pltpu.VMEM((2, YW, CS), jnp.float32),
        pltpu.VMEM((POOL, POOL, CS), jnp.float32),
        pltpu.SemaphoreType.DMA,
        pltpu.SemaphoreType.DMA,
    ],
)(_sc_body)


def _tc_body(rois_ref, feat_ref, out_ref):
    b = pl.program_id(0)
    n = pl.program_id(1)

    def rd(k):
        return rois_ref[b, n, k]

    min_x, min_y, max_x, max_y = rd(0), rd(1), rd(2), rd(3)
    dx = (max_x - min_x) // POOL
    dy = (max_y - min_y) // POOL

    for h in range(POOL):
        wy = 16 if h + 1 < POOL else 24
        assert (W - wy) % 8 == 0
        y1 = min_y + h * dy
        y2 = min_y + (h + 1) * dy if h + 1 < POOL else max_y
        y1c = jnp.minimum((y1 // 8) * 8, jnp.int32(W - wy))
        y1c = pl.multiple_of(y1c, 8)
        for w in range(POOL):
            wx = 5 if w + 1 < POOL else 10
            x1 = min_x + w * dx
            x2 = min_x + (w + 1) * dx if w + 1 < POOL else max_x
            x1c = jnp.minimum(x1, jnp.int32(H - wx))
            win = feat_ref[0, pl.ds(x1c, wx), pl.ds(y1c, wy), CS:]
            xg = lax.broadcasted_iota(jnp.int32, (wx, wy, 1), 0) + x1c
            yg = lax.broadcasted_iota(jnp.int32, (wx, wy, 1), 1) + y1c
            m = (xg >= x1) & (xg < x2) & (yg >= y1) & (yg < y2)
            cell = jnp.max(jnp.where(m, win, -jnp.inf), axis=(0, 1))
            out_ref[0, 0, h, w, :] = cell


_tc_pool = pl.pallas_call(
    _tc_body,
    grid_spec=pltpu.PrefetchScalarGridSpec(
        num_scalar_prefetch=1,
        grid=(B, N),
        in_specs=[
            pl.BlockSpec((1, H, W, C), lambda b, n, rois: (b, 0, 0, 0)),
        ],
        out_specs=pl.BlockSpec(
            (1, 1, POOL, POOL, CTC), lambda b, n, rois: (b, n, 0, 0, 0)
        ),
    ),
    out_shape=jax.ShapeDtypeStruct((B, N, POOL, POOL, CTC), jnp.float32),
)


def kernel(features, rois):
    rois_pad = jnp.zeros((B * N, LANES), jnp.float32)
    rois_pad = rois_pad.at[:, :4].set(rois.reshape(B * N, 4)).reshape(-1)
    out_sc = _sc_pool(features, rois_pad)
    out_tc = _tc_pool(rois.astype(jnp.int32), features)
    return jnp.concatenate([out_sc, out_tc], axis=-1)


# speedup vs baseline: 1.0804x; 1.0180x over previous
import functools

import jax
import jax.numpy as jnp
from jax import lax
from jax.experimental import pallas as pl
from jax.experimental.pallas import tpu as pltpu
from jax.experimental.pallas import tpu_sc as plsc

POOL = 7
C = 768
H = 56
W = 56
B = 2
N = 16
LANES = 16
YW = 48
GK = 16
CS = 256
CTC = C - CS
WIN = 16
WINY = 24

_GROUPS = []
_base = 0
while _base < CS:
    _nv = min(GK, (CS - _base) // LANES)
    _GROUPS.append((_base, _nv))
    _base += _nv * LANES


def _sc_body(feat_hbm, rois_hbm, out_hbm, rois_v, row_v, acc_v, sem0, sem1):
    b = lax.axis_index("c")
    n = lax.axis_index("s")
    wid = b * N + n

    pltpu.sync_copy(rois_hbm.at[pl.ds((wid // 2) * 8, LANES)], rois_v)
    vf = rois_v[...]
    odd = (wid % 2) == 1

    def _lane(j):
        return jnp.where(odd, vf[4 + j], vf[j]).astype(jnp.int32)

    min_x, min_y, max_x, max_y = _lane(0), _lane(1), _lane(2), _lane(3)
    dx = (max_x - min_x) // POOL
    dy = (max_y - min_y) // POOL

    y0 = jnp.minimum((min_y // 8) * 8, jnp.int32(W - YW))
    dmy = min_y - y0

    neg_inf = jnp.full((LANES,), -jnp.inf, jnp.float32)

    for h in range(POOL):
        for w in range(POOL):

            def _init(i, carry, h=h, w=w):
                acc_v[h, w, pl.ds(i * LANES, LANES)] = neg_inf
                return carry

            lax.fori_loop(0, CS // LANES, _init, jnp.int32(0))

    sems = (sem0, sem1)

    def _start(x, p):
        pltpu.async_copy(
            feat_hbm.at[b, x, pl.ds(y0, YW), pl.ds(0, CS)],
            row_v.at[p],
            sems[p],
        )

    def _wait(p):
        pltpu.make_async_copy(
            feat_hbm.at[0, 0, pl.ds(0, YW), pl.ds(0, CS)],
            row_v.at[p],
            sems[p],
        ).wait()

    def _compute(x, p):
        xr = x - min_x
        w_idx = jnp.int32(0)
        for k in range(1, POOL):
            w_idx = w_idx + (xr >= k * dx).astype(jnp.int32)
        for h in range(POOL):
            o1 = dmy + h * dy
            o2 = dmy + ((h + 1) * dy if h + 1 < POOL else max_y - min_y)
            for gbase, nv in _GROUPS:
                carries = tuple(
                    acc_v[h, w_idx, pl.ds(gbase + j * LANES, LANES)]
                    for j in range(nv)
                )

                def _ybody(y, cs2, gbase=gbase, nv=nv):
                    return tuple(
                        jnp.maximum(
                            cs2[j], row_v[p, y, pl.ds(gbase + j * LANES, LANES)]
                        )
                        for j in range(nv)
                    )

                carries = lax.fori_loop(o1, o2, _ybody, carries)
                for j in range(nv):
                    acc_v[h, w_idx, pl.ds(gbase + j * LANES, LANES)] = carries[j]

    nx = max_x - min_x
    _start(min_x, 0)

    def _pair(k, carry):
        x0 = min_x + 2 * k
        has1 = x0 + 1 < max_x

        @pl.when(has1)
        def _():
            _start(x0 + 1, 1)

        _wait(0)
        _compute(x0, 0)

        @pl.when(has1)
        def _():
            @pl.when(x0 + 2 < max_x)
            def _():
                _start(x0 + 2, 0)

            _wait(1)
            _compute(x0 + 1, 1)

        return carry

    lax.fori_loop(0, (nx + 1) // 2, _pair, jnp.int32(0))

    pltpu.sync_copy(acc_v, out_hbm.at[b, n])


_mesh = plsc.VectorSubcoreMesh(core_axis_name="c", subcore_axis_name="s")

_sc_pool = functools.partial(
    pl.kernel,
    mesh=_mesh,
    out_type=jax.ShapeDtypeStruct((B, N, POOL, POOL, CS), jnp.float32),
    scratch_types=[
        pltpu.VMEM((LANES,), jnp.float32),
        pltpu.VMEM((2, YW, CS), jnp.float32),
        pltpu.VMEM((POOL, POOL, CS), jnp.float32),
        pltpu.SemaphoreType.DMA,
        pltpu.SemaphoreType.DMA,
    ],
)(_sc_body)


def _tc_body(rois_ref, feat_ref, out_ref):
    b = pl.program_id(0)
    n = pl.program_id(1)

    def rd(k):
        return rois_ref[b, n, k].astype(jnp.int32)

    min_x, min_y, max_x, max_y = rd(0), rd(1), rd(2), rd(3)
    dx = (max_x - min_x) // POOL
    dy = (max_y - min_y) // POOL

    for h in range(POOL):
        wy = 16 if h + 1 < POOL else 24
        assert (W - wy) % 8 == 0
        y1 = min_y + h * dy
        y2 = min_y + (h + 1) * dy if h + 1 < POOL else max_y
        y1c = jnp.minimum((y1 // 8) * 8, jnp.int32(W - wy))
        y1c = pl.multiple_of(y1c, 8)
        for w in range(POOL):
            wx = 5 if w + 1 < POOL else 10
            x1 = min_x + w * dx
            x2 = min_x + (w + 1) * dx if w + 1 < POOL else max_x
            x1c = jnp.minimum(x1, jnp.int32(H - wx))
            win = feat_ref[0, pl.ds(x1c, wx), pl.ds(y1c, wy), CS:]
            xg = lax.broadcasted_iota(jnp.int32, (wx, wy, 1), 0) + x1c
            yg = lax.broadcasted_iota(jnp.int32, (wx, wy, 1), 1) + y1c
            m = (xg >= x1) & (xg < x2) & (yg >= y1) & (yg < y2)
            cell = jnp.max(jnp.where(m, win, -jnp.inf), axis=(0, 1))
            out_ref[0, 0, h, w, :] = cell


_tc_pool = pl.pallas_call(
    _tc_body,
    grid_spec=pltpu.PrefetchScalarGridSpec(
        num_scalar_prefetch=1,
        grid=(B, N),
        in_specs=[
            pl.BlockSpec((1, H, W, C), lambda b, n, rois: (b, 0, 0, 0)),
        ],
        out_specs=pl.BlockSpec(
            (1, 1, POOL, POOL, CTC), lambda b, n, rois: (b, n, 0, 0, 0)
        ),
    ),
    out_shape=jax.ShapeDtypeStruct((B, N, POOL, POOL, CTC), jnp.float32),
)


def kernel(features, rois):
    out_sc = _sc_pool(features, rois.reshape(-1))
    out_tc = _tc_pool(rois, features)
    return jnp.concatenate([out_sc, out_tc], axis=-1)
